# Initial kernel scaffold; baseline (speedup 1.0000x reference)
import jax
import jax.numpy as jnp
from jax.experimental import pallas as pl


def kernel(subseq_set, segment_ids, target_idx, w_i, item_emb, W_U_w, W_U_b, W_I_w, W_I_b):
    raise NotImplementedError("write your pallas kernel here")


# speedup vs baseline: 3.4542x; 3.4542x over previous
import functools

import jax
import jax.numpy as jnp
from jax import lax
from jax.experimental import pallas as pl
from jax.experimental.pallas import tpu as pltpu
from jax.experimental.pallas import tpu_sc as plsc

B = 4096
TOTAL_K = 16384
D = 64

NC = 2
NS = 16
NW = NC * NS
K_PER_W = TOTAL_K // NW
CHUNK = 128
NCHUNK = K_PER_W // CHUNK
B_PER_W = B // NW
B_PER_S = B // NS
CNT_W = 16


def _sc_gather_segsum(last_idx, seg_ids, tgt_idx, ones_blk, z_sums, z_cnts,
                      item_emb):
    mesh = plsc.VectorSubcoreMesh(core_axis_name="c", subcore_axis_name="s")

    @functools.partial(
        pl.kernel,
        mesh=mesh,
        compiler_params=pltpu.CompilerParams(use_tc_tiling_on_sc=False),
        out_type=[
            jax.ShapeDtypeStruct((NC * B, D), jnp.float32),
            jax.ShapeDtypeStruct((NC * B, CNT_W), jnp.float32),
            jax.ShapeDtypeStruct((B, D), jnp.float32),
        ],
        scratch_types=[
            pltpu.VMEM((NCHUNK, CHUNK), jnp.int32),
            pltpu.VMEM((NCHUNK, CHUNK), jnp.int32),
            pltpu.VMEM((NCHUNK, CHUNK, D), jnp.float32),
            pltpu.VMEM((CHUNK, CNT_W), jnp.float32),
            pltpu.VMEM((B_PER_W,), jnp.int32),
            pltpu.VMEM((B_PER_W, D), jnp.float32),
            pltpu.VMEM_SHARED((B, D), jnp.float32),
            pltpu.VMEM_SHARED((B, CNT_W), jnp.float32),
            pltpu.SemaphoreType.DMA,
        ],
    )
    def k(last_hbm, seg_hbm, tidx_hbm, ones_hbm, z64_hbm, z16_hbm, item_hbm,
          sums_out, cnts_out, tgt_out,
          idx_v, seg_v, rows_v, ones_v, tidx_v, trows_v, sums_sh, cnts_sh,
          sem):
        c = lax.axis_index("c")
        s = lax.axis_index("s")
        wid = s * NC + c

        pltpu.sync_copy(z64_hbm, sums_sh.at[pl.ds(s * B_PER_S, B_PER_S)])
        pltpu.sync_copy(z16_hbm, cnts_sh.at[pl.ds(s * B_PER_S, B_PER_S)])

        pltpu.sync_copy(last_hbm.at[wid], idx_v)
        pltpu.sync_copy(seg_hbm.at[wid], seg_v)
        pltpu.sync_copy(ones_hbm, ones_v)
        pltpu.sync_copy(tidx_hbm.at[wid], tidx_v)

        copies = [
            pltpu.async_copy(item_hbm.at[idx_v.at[j]], rows_v.at[j], sem)
            for j in range(NCHUNK)
        ]
        tcopy = pltpu.async_copy(item_hbm.at[tidx_v], trows_v, sem)
        for cp in copies:
            cp.wait()
        tcopy.wait()
        pltpu.sync_copy(trows_v, tgt_out.at[pl.ds(wid * B_PER_W, B_PER_W)])

        plsc.subcore_barrier()

        for j in range(NCHUNK):
            pltpu.sync_copy(rows_v.at[j], sums_sh.at[seg_v.at[j]], add=True)
            pltpu.sync_copy(ones_v, cnts_sh.at[seg_v.at[j]], add=True)

        plsc.subcore_barrier()

        row0 = c * B + s * B_PER_S
        pltpu.sync_copy(sums_sh.at[pl.ds(s * B_PER_S, B_PER_S)],
                        sums_out.at[pl.ds(row0, B_PER_S)])
        pltpu.sync_copy(cnts_sh.at[pl.ds(s * B_PER_S, B_PER_S)],
                        cnts_out.at[pl.ds(row0, B_PER_S)])

    return k(last_idx, seg_ids, tgt_idx, ones_blk, z_sums, z_cnts, item_emb)


def _tc_loss_body(ps_ref, pc_ref, tgt_ref, wi_ref, wu_ref, wub_ref,
                  wiw_ref, wib_ref, out_ref):
    sums = ps_ref[0:B, :] + ps_ref[B:2 * B, :]
    cnt_blk = pc_ref[0:B, :] + pc_ref[B:2 * B, :]
    counts = jnp.sum(cnt_blk, axis=1, keepdims=True) * (1.0 / CNT_W)
    mean = sums / jnp.maximum(counts, 1.0)
    few = mean + lax.dot_general(mean, wu_ref[...],
                                 (((1,), (1,)), ((), ())),
                                 preferred_element_type=jnp.float32)
    few = few + jnp.minimum(counts, 1.0) * wub_ref[...]
    pred = lax.dot_general(few, wiw_ref[...],
                           (((1,), (1,)), ((), ())),
                           preferred_element_type=jnp.float32) + wib_ref[...]
    diff = pred - tgt_ref[...]
    loss = jnp.sum(wi_ref[...] * diff * diff) * (1.0 / (B * D))
    out_ref[...] = jnp.reshape(loss, (1, 1))


def kernel(subseq_set, segment_ids, target_idx, w_i, item_emb, W_U_w, W_U_b,
           W_I_w, W_I_b):
    last_idx = subseq_set[:, -1].reshape(NW, NCHUNK, CHUNK)
    seg = segment_ids.reshape(NW, NCHUNK, CHUNK)
    tidx = target_idx.reshape(NW, B_PER_W)
    ones_blk = jnp.ones((CHUNK, CNT_W), jnp.float32)
    z_sums = jnp.zeros((B_PER_S, D), jnp.float32)
    z_cnts = jnp.zeros((B_PER_S, CNT_W), jnp.float32)

    part_sums, part_cnts, tgt = _sc_gather_segsum(
        last_idx, seg, tidx, ones_blk, z_sums, z_cnts, item_emb)

    loss = pl.pallas_call(
        _tc_loss_body,
        out_shape=jax.ShapeDtypeStruct((1, 1), jnp.float32),
    )(part_sums, part_cnts, tgt, w_i, W_U_w, W_U_b.reshape(1, D),
      W_I_w, W_I_b.reshape(1, D))
    return loss[0, 0]


# speedup vs baseline: 4.3572x; 1.2614x over previous
import functools

import jax
import jax.numpy as jnp
from jax import lax
from jax.experimental import pallas as pl
from jax.experimental.pallas import tpu as pltpu
from jax.experimental.pallas import tpu_sc as plsc

B = 4096
TOTAL_K = 16384
D = 64
VOCAB = 1000000

NC = 2
NS = 16
NW = NC * NS

BLK = 128
BASE_MAX = VOCAB - BLK
U = 4
CNT_W = 16


def _build_worklist(sid, ipw):
    n = sid.shape[0]
    maxit = ipw + 4
    jj = jnp.arange(n, dtype=jnp.int32)
    w_of = jj // ipw
    jloc = jj % ipw
    bid = sid >> 7
    newb = (jloc == 0) | (bid != jnp.roll(bid, 1))
    runstart = lax.cummax(jnp.where(newb, jj, 0))
    r = jj - runstart
    step = (newb | (r % U == 0)).astype(jnp.int32)
    it_glob = jnp.cumsum(step) - 1
    it_loc = (it_glob - jnp.take(it_glob, w_of * ipw)).astype(jnp.int32)
    sl = r % U
    n_it = it_loc.reshape(NW, ipw)[:, -1] + 1
    n_it = n_it + (n_it & 1)
    n_pair = jnp.concatenate([(n_it // 2).astype(jnp.int32),
                              jnp.zeros((16,), jnp.int32)])
    col = sid - jnp.minimum(bid * BLK, BASE_MAX)
    ablk = jnp.zeros((NW, maxit + 16), jnp.int32).at[w_of, it_loc].set(bid)
    wl = maxit * U
    acol = jnp.zeros((NW, wl + 16), jnp.int32
                     ).at[w_of, it_loc * U + sl].set(col)
    adst = jnp.full((NW, wl + 16), ipw, jnp.int32
                    ).at[w_of, it_loc * U + sl].set(jloc)
    return ablk, acol, adst, n_pair


def _stream_body(ipw, tab_hbm, ablk_v, acol_v, adst_v, npair_v, rows_v,
                 blkA, blkB, semA, semB, wid):
    feats = lax.iota(jnp.int32, 16)

    def sread(ref, i):
        return ref[pl.ds(i, 16)][0]

    def base_at(t):
        g = sread(ablk_v, t)
        return pl.multiple_of(jnp.minimum(g * BLK, BASE_MAX), BLK)

    def fetch(t, blk, sem):
        pltpu.async_copy(tab_hbm.at[:, pl.ds(base_at(t), BLK)], blk, sem)

    def wait(t, blk, sem):
        pltpu.make_async_copy(
            tab_hbm.at[:, pl.ds(base_at(t), BLK)], blk, sem).wait()

    def extract(blk, t):
        for u in range(U):
            col = sread(acol_v, t * U + u)
            dst = sread(adst_v, t * U + u)
            for q in range(4):
                v = plsc.load_gather(blk, [16 * q + feats,
                                           jnp.full((16,), col, jnp.int32)])
                rows_v[dst, pl.ds(16 * q, 16)] = v

    n_pairs = sread(npair_v, wid)
    fetch(0, blkA, semA)
    fetch(1, blkB, semB)

    def pair(i, carry):
        tA = 2 * i
        wait(tA, blkA, semA)
        extract(blkA, tA)
        fetch(tA + 2, blkA, semA)
        tB = 2 * i + 1
        wait(tB, blkB, semB)
        extract(blkB, tB)
        fetch(tB + 2, blkB, semB)
        return carry

    lax.fori_loop(0, n_pairs, pair, jnp.int32(0))
    wait(2 * n_pairs, blkA, semA)
    wait(2 * n_pairs + 1, blkB, semB)


IPW_M = TOTAL_K // NW
NBATCH = IPW_M // 128
MAXIT_M = IPW_M + 4
WL_M = MAXIT_M * U


def _sc_main(ablk, acol, adst, n_pair, tableT):
    mesh = plsc.VectorSubcoreMesh(core_axis_name="c", subcore_axis_name="s")

    @functools.partial(
        pl.kernel,
        mesh=mesh,
        compiler_params=pltpu.CompilerParams(use_tc_tiling_on_sc=True,
                                             needs_layout_passes=False),
        out_type=jax.ShapeDtypeStruct((TOTAL_K, D), jnp.float32),
        scratch_types=[
            pltpu.VMEM((MAXIT_M + 16,), jnp.int32),
            pltpu.VMEM((WL_M + 16,), jnp.int32),
            pltpu.VMEM((WL_M + 16,), jnp.int32),
            pltpu.VMEM((NW + 16,), jnp.int32),
            pltpu.VMEM((D, BLK), jnp.float32),
            pltpu.VMEM((D, BLK), jnp.float32),
            pltpu.VMEM((IPW_M + 1, D), jnp.float32),
            pltpu.SemaphoreType.DMA,
            pltpu.SemaphoreType.DMA,
        ],
    )
    def k(ablk_hbm, acol_hbm, adst_hbm, npair_hbm, tab_hbm, rows_out,
          ablk_v, acol_v, adst_v, npair_v, blkA, blkB, rows_v, semA, semB):
        c = lax.axis_index("c")
        s = lax.axis_index("s")
        wid = s * NC + c

        pltpu.sync_copy(ablk_hbm.at[wid], ablk_v)
        pltpu.sync_copy(acol_hbm.at[wid], acol_v)
        pltpu.sync_copy(adst_hbm.at[wid], adst_v)
        pltpu.sync_copy(npair_hbm, npair_v)

        _stream_body(IPW_M, tab_hbm, ablk_v, acol_v, adst_v, npair_v,
                     rows_v, blkA, blkB, semA, semB, wid)

        pltpu.sync_copy(rows_v.at[pl.ds(0, IPW_M)],
                        rows_out.at[pl.ds(wid * IPW_M, IPW_M)])

    return k(ablk, acol, adst, n_pair, tableT)


def _sc_segsum(rows, mdst3, ones_blk, z_acc, z_cnt):
    mesh = plsc.VectorSubcoreMesh(core_axis_name="c", subcore_axis_name="s")

    @functools.partial(
        pl.kernel,
        mesh=mesh,
        compiler_params=pltpu.CompilerParams(use_tc_tiling_on_sc=False),
        out_type=[
            jax.ShapeDtypeStruct((NC * B, D), jnp.float32),
            jax.ShapeDtypeStruct((NC * B, CNT_W), jnp.float32),
        ],
        scratch_types=[
            pltpu.VMEM((NBATCH, 128), jnp.int32),
            pltpu.VMEM((IPW_M, D), jnp.float32),
            pltpu.VMEM((128, CNT_W), jnp.float32),
            pltpu.VMEM_SHARED((B, D), jnp.float32),
            pltpu.VMEM_SHARED((B, CNT_W), jnp.float32),
        ],
    )
    def k(rows_hbm, mdst3_hbm, ones_hbm, zacc_hbm, zcnt_hbm,
          acc_out, cnts_out, mdst_v, rows_v, ones_v, acc_sh, cnts_sh):
        c = lax.axis_index("c")
        s = lax.axis_index("s")
        wid = s * NC + c
        psz = B // NS

        pltpu.sync_copy(zacc_hbm, acc_sh.at[pl.ds(s * psz, psz)])
        pltpu.sync_copy(zcnt_hbm, cnts_sh.at[pl.ds(s * psz, psz)])
        pltpu.sync_copy(mdst3_hbm.at[wid], mdst_v)
        pltpu.sync_copy(rows_hbm.at[pl.ds(wid * IPW_M, IPW_M)], rows_v)
        pltpu.sync_copy(ones_hbm, ones_v)

        plsc.subcore_barrier()

        for kk in range(NBATCH):
            pltpu.sync_copy(rows_v.at[pl.ds(kk * 128, 128)],
                            acc_sh.at[mdst_v.at[kk]], add=True)
            pltpu.sync_copy(ones_v, cnts_sh.at[mdst_v.at[kk]], add=True)

        plsc.subcore_barrier()

        a0 = c * B + s * psz
        pltpu.sync_copy(acc_sh.at[pl.ds(s * psz, psz)],
                        acc_out.at[pl.ds(a0, psz)])
        pltpu.sync_copy(cnts_sh.at[pl.ds(s * psz, psz)],
                        cnts_out.at[pl.ds(a0, psz)])

    return k(rows, mdst3, ones_blk, z_acc, z_cnt)


IPW_T = B // NW
MAXIT_T = IPW_T + 4
WL_T = MAXIT_T * U


def _sc_target(ablk, acol, adst, n_pair, tableT):
    mesh = plsc.VectorSubcoreMesh(core_axis_name="c", subcore_axis_name="s")

    @functools.partial(
        pl.kernel,
        mesh=mesh,
        compiler_params=pltpu.CompilerParams(use_tc_tiling_on_sc=True,
                                             needs_layout_passes=False),
        out_type=jax.ShapeDtypeStruct((B, D), jnp.float32),
        scratch_types=[
            pltpu.VMEM((MAXIT_T + 16,), jnp.int32),
            pltpu.VMEM((WL_T + 16,), jnp.int32),
            pltpu.VMEM((WL_T + 16,), jnp.int32),
            pltpu.VMEM((NW + 16,), jnp.int32),
            pltpu.VMEM((D, BLK), jnp.float32),
            pltpu.VMEM((D, BLK), jnp.float32),
            pltpu.VMEM((IPW_T + 1, D), jnp.float32),
            pltpu.SemaphoreType.DMA,
            pltpu.SemaphoreType.DMA,
        ],
    )
    def k(ablk_hbm, acol_hbm, adst_hbm, npair_hbm, tab_hbm, tgt_out,
          ablk_v, acol_v, adst_v, npair_v, blkA, blkB, rows_v, semA, semB):
        c = lax.axis_index("c")
        s = lax.axis_index("s")
        wid = s * NC + c

        pltpu.sync_copy(ablk_hbm.at[wid], ablk_v)
        pltpu.sync_copy(acol_hbm.at[wid], acol_v)
        pltpu.sync_copy(adst_hbm.at[wid], adst_v)
        pltpu.sync_copy(npair_hbm, npair_v)

        _stream_body(IPW_T, tab_hbm, ablk_v, acol_v, adst_v, npair_v,
                     rows_v, blkA, blkB, semA, semB, wid)

        pltpu.sync_copy(rows_v.at[pl.ds(0, IPW_T)],
                        tgt_out.at[pl.ds(wid * IPW_T, IPW_T)])

    return k(ablk, acol, adst, n_pair, tableT)


def _tc_loss_body(acc_ref, pc_ref, tgt_ref, wi_ref, wu_ref, wub_ref,
                  wiw_ref, wib_ref, out_ref):
    sums = acc_ref[0:B, :] + acc_ref[B:2 * B, :]
    cnt_blk = pc_ref[0:B, :] + pc_ref[B:2 * B, :]
    counts = jnp.sum(cnt_blk, axis=1, keepdims=True) * (1.0 / CNT_W)
    mean = sums / jnp.maximum(counts, 1.0)
    few = mean + lax.dot_general(mean, wu_ref[...],
                                 (((1,), (1,)), ((), ())),
                                 preferred_element_type=jnp.float32)
    few = few + jnp.minimum(counts, 1.0) * wub_ref[...]
    pred = lax.dot_general(few, wiw_ref[...],
                           (((1,), (1,)), ((), ())),
                           preferred_element_type=jnp.float32) + wib_ref[...]
    diff = pred - tgt_ref[...]
    loss = jnp.sum(wi_ref[...] * diff * diff) * (1.0 / (B * D))
    out_ref[...] = jnp.reshape(loss, (1, 1))


def kernel(subseq_set, segment_ids, target_idx, w_i, item_emb, W_U_w, W_U_b,
           W_I_w, W_I_b):
    tableT = item_emb.T

    last_idx = subseq_set[:, -1]
    order_m = jnp.argsort(last_idx)
    sid_m = last_idx[order_m]
    mdst3 = segment_ids[order_m].reshape(NW, NBATCH, 128)
    ablk_m, acol_m, adst_m, npair_m = _build_worklist(sid_m, IPW_M)

    order_t = jnp.argsort(target_idx)
    sid_t = target_idx[order_t]
    ablk_t, acol_t, adst_t, npair_t = _build_worklist(sid_t, IPW_T)

    ones_blk = jnp.ones((128, CNT_W), jnp.float32)
    z_acc = jnp.zeros((B // NS, D), jnp.float32)
    z_cnt = jnp.zeros((B // NS, CNT_W), jnp.float32)

    rows = _sc_main(ablk_m, acol_m, adst_m, npair_m, tableT)
    acc, part_cnts = _sc_segsum(rows, mdst3, ones_blk, z_acc, z_cnt)
    tgt_sorted = _sc_target(ablk_t, acol_t, adst_t, npair_t, tableT)
    inv_t = jnp.argsort(order_t)
    tgt = jnp.take(tgt_sorted, inv_t, axis=0)

    loss = pl.pallas_call(
        _tc_loss_body,
        out_shape=jax.ShapeDtypeStruct((1, 1), jnp.float32),
    )(acc, part_cnts, tgt, w_i, W_U_w, W_U_b.reshape(1, D),
      W_I_w, W_I_b.reshape(1, D))
    return loss[0, 0]
